# baseline (speedup 1.0000x reference)
import functools

import jax
import jax.numpy as jnp
from jax import lax
from jax.experimental import pallas as pl
from jax.experimental.pallas import tpu as pltpu
from jax.experimental.pallas import tpu_sc as plsc

NUM_CORES = 2
NUM_SUBCORES = 16
NUM_WORKERS = NUM_CORES * NUM_SUBCORES
CHUNK = 1024


@functools.lru_cache(maxsize=None)
def _make_gather(B, V, D):
    b_per_w = B // NUM_WORKERS
    n_chunks = b_per_w // CHUNK
    mesh = plsc.VectorSubcoreMesh(
        core_axis_name="c",
        subcore_axis_name="s",
        num_cores=NUM_CORES,
        num_subcores=NUM_SUBCORES,
    )

    @functools.partial(
        pl.kernel,
        out_type=jax.ShapeDtypeStruct((B, D), jnp.float32),
        mesh=mesh,
        scratch_types=[
            pltpu.VMEM((CHUNK,), jnp.int32),
            pltpu.VMEM((CHUNK, D), jnp.float32),
            pltpu.SemaphoreType.DMA,
        ],
        compiler_params=pltpu.CompilerParams(use_tc_tiling_on_sc=False),
    )
    def gather_kernel(idx_hbm, table_hbm, out_hbm, idx_v, rows_v, sem):
        wid = lax.axis_index("s") * NUM_CORES + lax.axis_index("c")
        base = wid * b_per_w

        def body(g, carry):
            off = base + g * CHUNK
            pltpu.sync_copy(idx_hbm.at[pl.ds(off, CHUNK)], idx_v)
            pltpu.async_copy(table_hbm.at[idx_v], rows_v, sem).wait()
            pltpu.sync_copy(rows_v, out_hbm.at[pl.ds(off, CHUNK)])
            return carry

        lax.fori_loop(0, n_chunks, body, 0)

    return gather_kernel


def kernel(indices, table):
    B0, H = indices.shape
    V, D = table.shape
    B = B0 * H
    idx_flat = indices.reshape(B).astype(jnp.int32)
    out = _make_gather(B, V, D)(idx_flat, table)
    return out.reshape(B0, H, D)


# speedup vs baseline: 1.0350x; 1.0350x over previous
import functools

import jax
import jax.numpy as jnp
from jax import lax
from jax.experimental import pallas as pl
from jax.experimental.pallas import tpu as pltpu
from jax.experimental.pallas import tpu_sc as plsc

NUM_CORES = 2
NUM_SUBCORES = 16
NUM_WORKERS = NUM_CORES * NUM_SUBCORES
CHUNK = 800
NBUF = 2


@functools.lru_cache(maxsize=None)
def _make_gather(B, V, D):
    b_per_w = B // NUM_WORKERS
    n_chunks = b_per_w // CHUNK
    assert n_chunks % NBUF == 0
    mesh = plsc.VectorSubcoreMesh(
        core_axis_name="c",
        subcore_axis_name="s",
        num_cores=NUM_CORES,
        num_subcores=NUM_SUBCORES,
    )

    @functools.partial(
        pl.kernel,
        out_type=jax.ShapeDtypeStruct((B, D), jnp.float32),
        mesh=mesh,
        scratch_types=[
            pltpu.VMEM((NBUF, CHUNK), jnp.int32),
            pltpu.VMEM((NBUF, CHUNK, D), jnp.float32),
        ]
        + [pltpu.SemaphoreType.DMA] * (3 * NBUF),
        compiler_params=pltpu.CompilerParams(use_tc_tiling_on_sc=False),
    )
    def gather_kernel(idx_hbm, table_hbm, out_hbm, idx_v, rows_v, *sems):
        sem_i = sems[0:NBUF]
        sem_g = sems[NBUF : 2 * NBUF]
        sem_o = sems[2 * NBUF : 3 * NBUF]
        wid = lax.axis_index("s") * NUM_CORES + lax.axis_index("c")
        base = wid * b_per_w

        for b in range(NBUF):
            off = base + b * CHUNK
            pltpu.async_copy(idx_hbm.at[pl.ds(off, CHUNK)], idx_v.at[b], sem_i[b])

        def outer_body(o, carry):
            for b in range(NBUF):
                c = o * NBUF + b
                off = base + c * CHUNK
                pltpu.make_async_copy(
                    idx_hbm.at[pl.ds(off, CHUNK)], idx_v.at[b], sem_i[b]
                ).wait()

                @pl.when(c >= NBUF)
                def _():
                    poff = base + (c - NBUF) * CHUNK
                    pltpu.make_async_copy(
                        rows_v.at[b], out_hbm.at[pl.ds(poff, CHUNK)], sem_o[b]
                    ).wait()

                pltpu.async_copy(table_hbm.at[idx_v.at[b]], rows_v.at[b], sem_g[b])
                pltpu.make_async_copy(
                    table_hbm.at[idx_v.at[b]], rows_v.at[b], sem_g[b]
                ).wait()

                pltpu.async_copy(rows_v.at[b], out_hbm.at[pl.ds(off, CHUNK)], sem_o[b])

                @pl.when(c + NBUF < n_chunks)
                def _():
                    noff = base + (c + NBUF) * CHUNK
                    pltpu.async_copy(
                        idx_hbm.at[pl.ds(noff, CHUNK)], idx_v.at[b], sem_i[b]
                    )

            return carry

        lax.fori_loop(0, n_chunks // NBUF, outer_body, 0)

        for b in range(NBUF):
            c = n_chunks - NBUF + b
            off = base + c * CHUNK
            pltpu.make_async_copy(
                rows_v.at[b], out_hbm.at[pl.ds(off, CHUNK)], sem_o[b]
            ).wait()

    return gather_kernel


def kernel(indices, table):
    B0, H = indices.shape
    V, D = table.shape
    B = B0 * H
    idx_flat = indices.reshape(B).astype(jnp.int32)
    out = _make_gather(B, V, D)(idx_flat, table)
    return out.reshape(B0, H, D)
